# baseline (speedup 1.0000x reference)
import functools

import jax
import jax.numpy as jnp
from jax import lax
from jax.experimental import pallas as pl
from jax.experimental.pallas import tpu as pltpu
from jax.experimental.pallas import tpu_sc as plsc

N = 4_000_000
BINS = 5
CHUNK = 8_000
NCHUNKS = N // CHUNK
NC, NS = 2, 16
NW = NC * NS
VPC = CHUNK // 16

_BASE = NCHUNKS // NW
_EXTRA = NCHUNKS % NW


def _sc_body(logits_hbm, targets_hbm, out_hbm, xbuf, ybuf, acc_s, acc_c):
    cid = lax.axis_index("c")
    sid = lax.axis_index("s")
    wid = sid * NC + cid
    nch = jnp.where(wid < _EXTRA, _BASE + 1, _BASE)
    start = wid * _BASE + jnp.minimum(wid, _EXTRA)

    zeros16 = jnp.zeros((16,), jnp.float32)
    for k in range(BINS):
        acc_s[pl.ds(k * 16, 16)] = zeros16
        acc_c[pl.ds(k * 16, 16)] = zeros16

    iota = lax.iota(jnp.int32, 16)
    xoff = 2 * iota + 1
    ones16 = jnp.ones((16,), jnp.float32)

    def chunk_body(j, carry):
        c = start + j
        pltpu.sync_copy(logits_hbm.at[pl.ds(c * (2 * CHUNK), 2 * CHUNK)], xbuf)
        pltpu.sync_copy(targets_hbm.at[pl.ds(c * CHUNK, CHUNK)], ybuf)

        def vec_body(v, carry2):
            y = ybuf[pl.ds(v * 16, 16)]
            x = plsc.load_gather(xbuf, [v * 32 + xoff])
            ax = jnp.abs(x)
            e = jnp.exp(-ax)
            r = 1.0 / (1.0 + e)
            s = jnp.where(x >= 0.0, r, 1.0 - r)
            g = jnp.abs(s - y)
            b = jnp.minimum((g * 5.0).astype(jnp.int32), 4)
            z = e / (e + 2.0)
            z2 = z * z
            p = 1.0 + z2 * (0.33333333 + z2 * (0.2 + z2 * (0.14285714 + z2 * 0.11111111)))
            softplus = 2.0 * z * p
            pe = jnp.maximum(x, 0.0) - x * y + softplus
            idx = b * 16 + iota
            plsc.addupdate_scatter(acc_s, [idx], pe)
            plsc.addupdate_scatter(acc_c, [idx], ones16)
            return carry2

        lax.fori_loop(0, VPC, vec_body, 0, unroll=2)
        return carry

    lax.fori_loop(0, nch, chunk_body, 0)
    pltpu.sync_copy(acc_s, out_hbm.at[pl.ds(wid * 160, 80)])
    pltpu.sync_copy(acc_c, out_hbm.at[pl.ds(wid * 160 + 80, 80)])


@jax.jit
def _ghm_sc(logits_flat, targets):
    mesh = plsc.VectorSubcoreMesh(
        core_axis_name="c", subcore_axis_name="s", num_cores=NC, num_subcores=NS
    )
    run = pl.kernel(
        _sc_body,
        out_type=jax.ShapeDtypeStruct((NW * 160,), jnp.float32),
        mesh=mesh,
        scratch_types=[
            pltpu.VMEM((2 * CHUNK,), jnp.float32),
            pltpu.VMEM((CHUNK,), jnp.float32),
            pltpu.VMEM((80,), jnp.float32),
            pltpu.VMEM((80,), jnp.float32),
        ],
        compiler_params=pltpu.CompilerParams(needs_layout_passes=False),
    )
    return run(logits_flat, targets)


def kernel(logits, targets):
    part = _ghm_sc(logits.reshape(-1), targets)
    p = part.reshape(NW, 2, BINS, 16)
    s_b = jnp.sum(p[:, 0], axis=(0, 2))
    c_b = jnp.sum(p[:, 1], axis=(0, 2))
    total = float(logits.size)
    w_b = jnp.where(c_b > 0, total / ((1.0 - 0.5) * c_b), 0.0)
    return jnp.sum(w_b * s_b) / targets.shape[0]


# speedup vs baseline: 1.0341x; 1.0341x over previous
import jax
import jax.numpy as jnp
from jax import lax
from jax.experimental import pallas as pl
from jax.experimental.pallas import tpu as pltpu
from jax.experimental.pallas import tpu_sc as plsc

N = 4_000_000
BINS = 5
CHUNK = 8_000
NCHUNKS = N // CHUNK
NC, NS = 2, 16
NW = NC * NS
VPC = CHUNK // 16

_BASE = NCHUNKS // NW
_EXTRA = NCHUNKS % NW

_LP = (
    2.554673020349618e-07, 0.9999670809438443, -0.49928504912226557,
    0.32722571497202635, -0.22316586411450423, 0.130833427976782,
    -0.05243753706207599, 0.01000928961639147,
)


def _sc_body(logits_hbm, targets_hbm, out_hbm, xbuf, ybuf, stage):
    cid = lax.axis_index("c")
    sid = lax.axis_index("s")
    wid = sid * NC + cid
    nch = jnp.where(wid < _EXTRA, _BASE + 1, _BASE)
    start = wid * _BASE + jnp.minimum(wid, _EXTRA)

    iota = lax.iota(jnp.int32, 16)
    xoff = 2 * iota + 1
    z16 = jnp.zeros((16,), jnp.float32)

    def chunk_body(j, carry):
        c = start + j
        pltpu.sync_copy(logits_hbm.at[pl.ds(c * (2 * CHUNK), 2 * CHUNK)], xbuf)
        pltpu.sync_copy(targets_hbm.at[pl.ds(c * CHUNK, CHUNK)], ybuf)

        @plsc.parallel_loop(0, VPC, unroll=4, carry=carry)
        def inner(v, acc):
            s_t, t1, t2, t3, t4, c1, c2, c3, c4 = acc
            y = ybuf[pl.ds(v * 16, 16)]
            x = plsc.load_gather(xbuf, [v * 32 + xoff])
            ax = jnp.abs(x)
            e = jnp.exp(-ax)
            r = 1.0 / (1.0 + e)
            yp = jnp.where(x >= 0.0, y, 1.0 - y)
            g = jnp.abs(r - yp)
            lp = _LP[7]
            for k in (6, 5, 4, 3, 2, 1, 0):
                lp = lp * e + _LP[k]
            pe = jnp.maximum(x, 0.0) - x * y + lp
            m1 = g >= 0.2
            m2 = g >= 0.4
            m3 = g >= 0.6
            m4 = g >= 0.8
            s_t = s_t + pe
            t1 = t1 + jnp.where(m1, pe, 0.0)
            t2 = t2 + jnp.where(m2, pe, 0.0)
            t3 = t3 + jnp.where(m3, pe, 0.0)
            t4 = t4 + jnp.where(m4, pe, 0.0)
            c1 = c1 + jnp.where(m1, 1.0, 0.0)
            c2 = c2 + jnp.where(m2, 1.0, 0.0)
            c3 = c3 + jnp.where(m3, 1.0, 0.0)
            c4 = c4 + jnp.where(m4, 1.0, 0.0)
            return (s_t, t1, t2, t3, t4, c1, c2, c3, c4)

        return inner

    acc0 = (z16,) * 9
    acc = lax.fori_loop(0, nch, chunk_body, acc0)
    for k in range(9):
        stage[pl.ds(k * 16, 16)] = acc[k]
    pltpu.sync_copy(stage, out_hbm.at[pl.ds(wid * 144, 144)])


@jax.jit
def _ghm_sc(logits_flat, targets):
    mesh = plsc.VectorSubcoreMesh(
        core_axis_name="c", subcore_axis_name="s", num_cores=NC, num_subcores=NS
    )
    run = pl.kernel(
        _sc_body,
        out_type=jax.ShapeDtypeStruct((NW * 144,), jnp.float32),
        mesh=mesh,
        scratch_types=[
            pltpu.VMEM((2 * CHUNK,), jnp.float32),
            pltpu.VMEM((CHUNK,), jnp.float32),
            pltpu.VMEM((144,), jnp.float32),
        ],
        compiler_params=pltpu.CompilerParams(needs_layout_passes=False),
    )
    return run(logits_flat, targets)


def kernel(logits, targets):
    part = _ghm_sc(logits.reshape(-1), targets)
    p = jnp.sum(part.reshape(NW, 9, 16), axis=(0, 2))
    s_tot, t1, t2, t3, t4, c1, c2, c3, c4 = (p[i] for i in range(9))
    n = float(targets.shape[0])
    s_b = jnp.stack([s_tot - t1, t1 - t2, t2 - t3, t3 - t4, t4])
    c_b = jnp.stack([n - c1, c1 - c2, c2 - c3, c3 - c4, c4])
    total = float(logits.size)
    w_b = jnp.where(c_b > 0, total / ((1.0 - 0.5) * c_b), 0.0)
    return jnp.sum(w_b * s_b) / targets.shape[0]


# speedup vs baseline: 17.8020x; 17.2143x over previous
import jax
import jax.numpy as jnp
from jax import lax
from jax.experimental import pallas as pl
from jax.experimental.pallas import tpu as pltpu
from jax.experimental.pallas import tpu_sc as plsc

N = 4_000_000
BINS = 5
CHUNK = 8_000
NCHUNKS = N // CHUNK
NC, NS = 2, 16
NW = NC * NS
VPC = CHUNK // 16

_BASE = NCHUNKS // NW
_EXTRA = NCHUNKS % NW

_LP = (
    2.554673020349618e-07, 0.9999670809438443, -0.49928504912226557,
    0.32722571497202635, -0.22316586411450423, 0.130833427976782,
    -0.05243753706207599, 0.01000928961639147,
)


def _sc_body(logits_hbm, targets_hbm, out_hbm, xbuf, ybuf, stage):
    cid = lax.axis_index("c")
    sid = lax.axis_index("s")
    wid = sid * NC + cid
    nch = jnp.where(wid < _EXTRA, _BASE + 1, _BASE)
    start = wid * _BASE + jnp.minimum(wid, _EXTRA)

    z16 = jnp.zeros((16,), jnp.float32)

    def chunk_body(j, carry):
        c = start + j
        pltpu.sync_copy(logits_hbm.at[pl.ds(c * CHUNK, CHUNK)], xbuf)
        pltpu.sync_copy(targets_hbm.at[pl.ds(c * CHUNK, CHUNK)], ybuf)

        @plsc.parallel_loop(0, VPC, unroll=4, carry=carry)
        def inner(v, acc):
            s_t, t1, t2, t3, t4, c1, c2, c3, c4 = acc
            y = ybuf[pl.ds(v * 16, 16)]
            x = xbuf[pl.ds(v * 16, 16)]
            ax = jnp.abs(x)
            e = jnp.exp(-ax)
            r = 1.0 / (1.0 + e)
            yp = jnp.where(x >= 0.0, y, 1.0 - y)
            g = jnp.abs(r - yp)
            lp = _LP[7]
            for k in (6, 5, 4, 3, 2, 1, 0):
                lp = lp * e + _LP[k]
            pe = jnp.maximum(x, 0.0) - x * y + lp
            m1 = g >= 0.2
            m2 = g >= 0.4
            m3 = g >= 0.6
            m4 = g >= 0.8
            s_t = s_t + pe
            t1 = t1 + jnp.where(m1, pe, 0.0)
            t2 = t2 + jnp.where(m2, pe, 0.0)
            t3 = t3 + jnp.where(m3, pe, 0.0)
            t4 = t4 + jnp.where(m4, pe, 0.0)
            c1 = c1 + jnp.where(m1, 1.0, 0.0)
            c2 = c2 + jnp.where(m2, 1.0, 0.0)
            c3 = c3 + jnp.where(m3, 1.0, 0.0)
            c4 = c4 + jnp.where(m4, 1.0, 0.0)
            return (s_t, t1, t2, t3, t4, c1, c2, c3, c4)

        return inner

    acc0 = (z16,) * 9
    acc = lax.fori_loop(0, nch, chunk_body, acc0)
    for k in range(9):
        stage[pl.ds(k * 16, 16)] = acc[k]
    pltpu.sync_copy(stage, out_hbm.at[pl.ds(wid * 144, 144)])


@jax.jit
def _ghm_sc(logits_flat, targets):
    mesh = plsc.VectorSubcoreMesh(
        core_axis_name="c", subcore_axis_name="s", num_cores=NC, num_subcores=NS
    )
    run = pl.kernel(
        _sc_body,
        out_type=jax.ShapeDtypeStruct((NW * 144,), jnp.float32),
        mesh=mesh,
        scratch_types=[
            pltpu.VMEM((CHUNK,), jnp.float32),
            pltpu.VMEM((CHUNK,), jnp.float32),
            pltpu.VMEM((144,), jnp.float32),
        ],
        compiler_params=pltpu.CompilerParams(
            needs_layout_passes=False, use_tc_tiling_on_sc=True
        ),
    )
    return run(logits_flat, targets)


def kernel(logits, targets):
    part = _ghm_sc(logits[:, 1], targets)
    p = jnp.sum(part.reshape(NW, 9, 16), axis=(0, 2))
    s_tot, t1, t2, t3, t4, c1, c2, c3, c4 = (p[i] for i in range(9))
    n = float(targets.shape[0])
    s_b = jnp.stack([s_tot - t1, t1 - t2, t2 - t3, t3 - t4, t4])
    c_b = jnp.stack([n - c1, c1 - c2, c2 - c3, c3 - c4, c4])
    total = float(logits.size)
    w_b = jnp.where(c_b > 0, total / ((1.0 - 0.5) * c_b), 0.0)
    return jnp.sum(w_b * s_b) / targets.shape[0]


# speedup vs baseline: 63.8513x; 3.5868x over previous
import jax
import jax.numpy as jnp
from jax import lax
from jax.experimental import pallas as pl
from jax.experimental.pallas import tpu as pltpu
from jax.experimental.pallas import tpu_sc as plsc

N = 4_000_000
BINS = 5
CHUNK = 16_000
NCHUNKS = N // CHUNK
NC, NS = 2, 16
NW = NC * NS
VPC = CHUNK // 16

_BASE = NCHUNKS // NW
_EXTRA = NCHUNKS % NW

_LT = (
    0.2876902085936771, -1.333342676597351, 0.8865566226237033,
    -0.7874456068595445, 0.8869645527936711, -0.9538804877894336,
)


BPC = CHUNK // 128


def _sc_body(x_hbm, y_hbm, out_hbm,
             xb0, yb0, xb1, yb1, acc_s, acc_c, sx0, sy0, sx1, sy1):
    cid = lax.axis_index("c")
    sid = lax.axis_index("s")
    wid = sid * NC + cid
    nch = jnp.where(wid < _EXTRA, _BASE + 1, _BASE)
    start = wid * _BASE + jnp.minimum(wid, _EXTRA)

    zeros16 = jnp.zeros((16,), jnp.float32)
    for k in range(BINS):
        acc_s[pl.ds(k * 16, 16)] = zeros16
        acc_c[pl.ds(k * 16, 16)] = zeros16

    iota = lax.iota(jnp.int32, 16)
    ones16 = jnp.ones((16,), jnp.float32)

    def start_dma(c, xb, yb, sx, sy):
        pltpu.async_copy(x_hbm.at[pl.ds(c * BPC, BPC), 1, :], xb, sx)
        pltpu.async_copy(y_hbm.at[pl.ds(c * CHUNK, CHUNK)], yb, sy)

    def wait_dma(c, xb, yb, sx, sy):
        pltpu.make_async_copy(x_hbm.at[pl.ds(c * BPC, BPC), 1, :], xb, sx).wait()
        pltpu.make_async_copy(y_hbm.at[pl.ds(c * CHUNK, CHUNK)], yb, sy).wait()

    def process(xb, yb):
        @plsc.parallel_loop(0, BPC, unroll=1)
        def _(blk):
            for k in range(8):
                y = yb[pl.ds(blk * 128 + k * 16, 16)]
                x = xb[blk, pl.ds(k * 16, 16)]
                e = jnp.exp(-x)
                r = 1.0 / (1.0 + e)
                g = jnp.abs(r - y)
                u = jnp.maximum(r, 1.0 - r) - 0.75
                p = jnp.float32(_LT[5])
                for j in (4, 3, 2, 1, 0):
                    p = p * u + _LT[j]
                pe = jnp.maximum(x, 0.0) - x * y + p
                b = jnp.minimum((g * 5.0).astype(jnp.int32), 4)
                idx = b * 16 + iota
                plsc.addupdate_scatter(acc_s, [idx], pe)
                plsc.addupdate_scatter(acc_c, [idx], ones16)

    start_dma(start, xb0, yb0, sx0, sy0)

    def chunk_body(j, carry):
        c = start + j
        even = (j % 2) == 0

        @pl.when(even)
        def _():
            @pl.when(j + 1 < nch)
            def _():
                start_dma(c + 1, xb1, yb1, sx1, sy1)
            wait_dma(c, xb0, yb0, sx0, sy0)
            process(xb0, yb0)

        @pl.when(jnp.logical_not(even))
        def _():
            @pl.when(j + 1 < nch)
            def _():
                start_dma(c + 1, xb0, yb0, sx0, sy0)
            wait_dma(c, xb1, yb1, sx1, sy1)
            process(xb1, yb1)

        return carry

    lax.fori_loop(0, nch, chunk_body, 0)
    pltpu.sync_copy(acc_s, out_hbm.at[pl.ds(wid * 160, 80)])
    pltpu.sync_copy(acc_c, out_hbm.at[pl.ds(wid * 160 + 80, 80)])


@jax.jit
def _ghm_sc(x, targets):
    mesh = plsc.VectorSubcoreMesh(
        core_axis_name="c", subcore_axis_name="s", num_cores=NC, num_subcores=NS
    )
    run = pl.kernel(
        _sc_body,
        out_type=jax.ShapeDtypeStruct((NW * 160,), jnp.float32),
        mesh=mesh,
        scratch_types=[
            pltpu.VMEM((BPC, 128), jnp.float32),
            pltpu.VMEM((CHUNK,), jnp.float32),
            pltpu.VMEM((BPC, 128), jnp.float32),
            pltpu.VMEM((CHUNK,), jnp.float32),
            pltpu.VMEM((80,), jnp.float32),
            pltpu.VMEM((80,), jnp.float32),
            pltpu.SemaphoreType.DMA,
            pltpu.SemaphoreType.DMA,
            pltpu.SemaphoreType.DMA,
            pltpu.SemaphoreType.DMA,
        ],
        compiler_params=pltpu.CompilerParams(needs_layout_passes=False),
    )
    return run(x, targets)


def kernel(logits, targets):
    x3 = logits.reshape(N // 128, 128, 2).transpose(0, 2, 1)
    part = _ghm_sc(x3, targets)
    p = part.reshape(NW, 2, BINS, 16)
    s_b = jnp.sum(p[:, 0], axis=(0, 2))
    c_b = jnp.sum(p[:, 1], axis=(0, 2))
    total = float(logits.size)
    w_b = jnp.where(c_b > 0, total / ((1.0 - 0.5) * c_b), 0.0)
    return jnp.sum(w_b * s_b) / targets.shape[0]
